# baseline (speedup 1.0000x reference)
import functools

import jax
import jax.numpy as jnp
from jax import lax
from jax.experimental import pallas as pl
from jax.experimental.pallas import tpu as pltpu
from jax.experimental.pallas import tpu_sc as plsc

EMBED_DIM = 128
NUM_WORKERS = 32
CHUNK = 200
NBUF = 4


def _embed(idx_flat, table):
    n = idx_flat.shape[0]
    per_w = n // NUM_WORKERS
    nchunks = per_w // CHUNK
    mesh = plsc.VectorSubcoreMesh(core_axis_name="c", subcore_axis_name="s")

    @functools.partial(
        pl.kernel,
        mesh=mesh,
        out_type=jax.ShapeDtypeStruct((n, EMBED_DIM), jnp.float32),
        compiler_params=pltpu.CompilerParams(use_tc_tiling_on_sc=True),
        scratch_types=(
            [pltpu.VMEM((CHUNK,), jnp.int32) for _ in range(NBUF)]
            + [pltpu.VMEM((CHUNK, EMBED_DIM), jnp.float32)
               for _ in range(NBUF)]
            + [pltpu.SemaphoreType.DMA for _ in range(2 * NBUF)]
        ),
    )
    def k(table_hbm, idx_hbm, out_hbm, *bufs):
        idxb = bufs[:NBUF]
        rows = bufs[NBUF:2 * NBUF]
        gsem = bufs[2 * NBUF:3 * NBUF]
        ssem = bufs[3 * NBUF:4 * NBUF]
        wid = lax.axis_index("s") * 2 + lax.axis_index("c")
        base = wid * per_w

        def load_idx(c, b):
            pltpu.sync_copy(idx_hbm.at[pl.ds(base + c * CHUNK, CHUNK)],
                            idxb[b])

        def gather(b):
            return pltpu.make_async_copy(
                table_hbm.at[idxb[b]], rows[b], gsem[b])

        def store(c, b):
            return pltpu.make_async_copy(
                rows[b], out_hbm.at[pl.ds(base + c * CHUNK, CHUNK)], ssem[b])

        for b in range(NBUF):
            load_idx(b, b)
            gather(b).start()

        @pl.loop(0, nchunks - NBUF, step=NBUF)
        def _(g):
            for b in range(NBUF):
                c = g + b
                gather(b).wait()
                store(c, b).start()
                load_idx(c + NBUF, b)
                store(c, b).wait()
                gather(b).start()

        for b in range(NBUF):
            c = nchunks - NBUF + b
            gather(b).wait()
            store(c, b).start()
        for b in range(NBUF):
            store(nchunks - NBUF + b, b).wait()

    return k(table, idx_flat)


def kernel(x, table):
    b, s = x.shape
    out = _embed(x.T.reshape(-1), table)
    return out.reshape(s, b, EMBED_DIM).transpose(1, 0, 2)


# speedup vs baseline: 1.0200x; 1.0200x over previous
import functools

import jax
import jax.numpy as jnp
from jax import lax
from jax.experimental import pallas as pl
from jax.experimental.pallas import tpu as pltpu
from jax.experimental.pallas import tpu_sc as plsc

EMBED_DIM = 128
NUM_WORKERS = 32
NUM_SUBCORES = 16
CHUNK = 200
NBUF = 2


def _embed(idx_flat, table):
    n = idx_flat.shape[0]
    per_w = n // NUM_WORKERS
    nchunks = per_w // CHUNK
    mesh = plsc.VectorSubcoreMesh(core_axis_name="c", subcore_axis_name="s")

    @functools.partial(
        pl.kernel,
        mesh=mesh,
        out_type=jax.ShapeDtypeStruct((n, EMBED_DIM), jnp.float32),
        compiler_params=pltpu.CompilerParams(use_tc_tiling_on_sc=True),
        scratch_types=(
            [pltpu.VMEM((CHUNK,), jnp.int32) for _ in range(NBUF)]
            + [pltpu.VMEM((CHUNK, EMBED_DIM), jnp.float32)
               for _ in range(NBUF)]
            + [pltpu.VMEM_SHARED(
                (NUM_SUBCORES, NBUF, CHUNK, EMBED_DIM), jnp.float32)]
            + [pltpu.SemaphoreType.DMA for _ in range(2 * NBUF)]
        ),
    )
    def k(table_hbm, idx_hbm, out_hbm, *bufs):
        idxb = bufs[:NBUF]
        rows = bufs[NBUF:2 * NBUF]
        shared = bufs[2 * NBUF]
        gsem = bufs[2 * NBUF + 1:3 * NBUF + 1]
        dsem = bufs[3 * NBUF + 1:4 * NBUF + 1]
        sid = lax.axis_index("s")
        wid = sid * 2 + lax.axis_index("c")
        base = wid * per_w

        def load_idx(c, b):
            pltpu.sync_copy(idx_hbm.at[pl.ds(base + c * CHUNK, CHUNK)],
                            idxb[b])

        def gather(b):
            return pltpu.make_async_copy(
                table_hbm.at[idxb[b]], rows[b], gsem[b])

        def hop(b):
            pltpu.sync_copy(rows[b], shared.at[sid, b])

        def dma_out(c, b):
            return pltpu.make_async_copy(
                shared.at[sid, b],
                out_hbm.at[pl.ds(base + c * CHUNK, CHUNK)], dsem[b])

        for b in range(NBUF):
            load_idx(b, b)
            gather(b).start()
        for b in range(NBUF):
            gather(b).wait()
            hop(b)
            dma_out(b, b).start()
            load_idx(b + NBUF, b)
            gather(b).start()

        @pl.loop(NBUF, nchunks - NBUF, step=NBUF)
        def _(g):
            for b in range(NBUF):
                c = g + b
                gather(b).wait()
                dma_out(c - NBUF, b).wait()
                hop(b)
                dma_out(c, b).start()
                load_idx(c + NBUF, b)
                gather(b).start()

        for b in range(NBUF):
            c = nchunks - NBUF + b
            gather(b).wait()
            dma_out(c - NBUF, b).wait()
            hop(b)
            dma_out(c, b).start()
        for b in range(NBUF):
            dma_out(nchunks - NBUF + b, b).wait()

    return k(table, idx_flat)


def kernel(x, table):
    b, s = x.shape
    out = _embed(x.T.reshape(-1), table)
    return out.reshape(s, b, EMBED_DIM).transpose(1, 0, 2)
